# Initial kernel scaffold; baseline (speedup 1.0000x reference)
import jax
import jax.numpy as jnp
from jax.experimental import pallas as pl


def kernel(S):
    raise NotImplementedError("write your pallas kernel here")


# speedup vs baseline: 157.9073x; 157.9073x over previous
import functools
import numpy as np
import jax
import jax.numpy as jnp
from jax import lax
from jax.experimental import pallas as pl
from jax.experimental.pallas import tpu as pltpu
from jax.experimental.pallas import tpu_sc as plsc

N_NBR = 8
V_THRESH = 0.001
B, H, W = 4, 64, 96
N = H * W
NC, NS, L = 2, 16, 16
NW = NC * NS
QPW = B * N // NW
NVEC = QPW // L
CHUNK = 16
DONE = N_NBR * QPW

_dyg, _dxg = np.mgrid[-(H - 1):H, -(W - 1):W]
_dxf = _dxg.ravel().astype(np.int64)
_dyf = _dyg.ravel().astype(np.int64)
_order = np.lexsort((_dyf * W + _dxf, _dxf * _dxf + _dyf * _dyf))
_dxs, _dys = _dxf[_order], _dyf[_order]
NCAND = len(_dxs)
_pad = (-NCAND) % CHUNK
_words = ((_dys + 64) << 9) | (_dxs + 96)
_words = np.concatenate([_words, np.full(_pad, (255 << 9) | 511, np.int64)])
NCTAB = len(_words)
NCHUNKS = NCTAB // CHUNK
_TAB_NP = _words.astype(np.int32)

SEG0 = 2
_SEG_ENDS = [6, 14, 30, 94, NCHUNKS]

_mesh = plsc.VectorSubcoreMesh(core_axis_name="c", subcore_axis_name="s")


@functools.partial(
    pl.kernel,
    out_type=(
        jax.ShapeDtypeStruct((B, 2, N_NBR, N), jnp.float32),
        jax.ShapeDtypeStruct((B, N_NBR, N), jnp.int32),
    ),
    mesh=_mesh,
    compiler_params=pltpu.CompilerParams(needs_layout_passes=False),
    scratch_types=[
        pltpu.VMEM((NCTAB,), jnp.int32),
        pltpu.VMEM((N,), jnp.float32),
        pltpu.VMEM((N_NBR * QPW,), jnp.int32),
        pltpu.VMEM((N_NBR * QPW,), jnp.float32),
        pltpu.VMEM((N_NBR * QPW,), jnp.float32),
    ],
)
def _dist_sc(s_hbm, tab_hbm, of_hbm, args_hbm, tab_v, s_v, args_l, ofx_l, ofy_l):
    wid = lax.axis_index("s") * NC + lax.axis_index("c")
    b = wid // (NW // B)
    part = wid % (NW // B)
    qbase = part * QPW

    pltpu.sync_copy(tab_hbm, tab_v)
    pltpu.sync_copy(s_hbm.at[b], s_v)

    zero16 = jnp.zeros((L,), jnp.int32)

    def init_body(i, _):
        args_l[pl.ds(i * L, L)] = zero16
        return 0

    lax.fori_loop(0, N_NBR * QPW // L, init_body, 0)

    def vec_body(v, _):
        qloc = v * L + lax.iota(jnp.int32, L)
        q = qbase + qloc
        qy = (q * 2731) >> 18
        qx = q - qy * W

        def chunk_body(c, sidx):
            base = c * CHUNK
            for k in range(CHUNK):
                widx = jnp.full((L,), base + k, jnp.int32)
                word = plsc.load_gather(tab_v, [widx])
                dxk = (word & 511) - 96
                dyk = (word >> 9) - 64
                nx = qx + dxk
                ny = qy + dyk
                inb = ((nx.astype(jnp.uint32) < W)
                       & (ny.astype(jnp.uint32) < H))
                j = ny * W + nx
                jc = jnp.where(inb, j, 0)
                sval = plsc.load_gather(s_v, [jc])
                found = inb & (sval > V_THRESH) & (sidx < DONE)
                plsc.store_scatter(args_l, [sidx], j, mask=found)
                sidx = sidx + jnp.where(found, QPW, 0)
            return sidx

        sidx = lax.fori_loop(0, SEG0, chunk_body, qloc)
        start = SEG0
        for end in _SEG_ENDS:
            left = jnp.max(jnp.where(sidx < DONE, 1, 0))
            stop = jnp.where(left > 0, end, start)
            sidx = lax.fori_loop(start, stop, chunk_body, sidx)
            start = end

        for k in range(N_NBR):
            jv = args_l[pl.ds(k * QPW + v * L, L)]
            jy = (jv * 2731) >> 18
            jx = jv - jy * W
            ofx_l[pl.ds(k * QPW + v * L, L)] = (jx - qx).astype(jnp.float32)
            ofy_l[pl.ds(k * QPW + v * L, L)] = (jy - qy).astype(jnp.float32)
        return 0

    lax.fori_loop(0, NVEC, vec_body, 0)

    for k in range(N_NBR):
        pltpu.sync_copy(args_l.at[pl.ds(k * QPW, QPW)],
                        args_hbm.at[b, k, pl.ds(qbase, QPW)])
        pltpu.sync_copy(ofx_l.at[pl.ds(k * QPW, QPW)],
                        of_hbm.at[b, 0, k, pl.ds(qbase, QPW)])
        pltpu.sync_copy(ofy_l.at[pl.ds(k * QPW, QPW)],
                        of_hbm.at[b, 1, k, pl.ds(qbase, QPW)])


@jax.jit
def kernel(S):
    Ofnum, args = _dist_sc(S.reshape(B, N), jnp.asarray(_TAB_NP))
    return (Ofnum, args)


# speedup vs baseline: 165.0690x; 1.0454x over previous
import functools
import numpy as np
import jax
import jax.numpy as jnp
from jax import lax
from jax.experimental import pallas as pl
from jax.experimental.pallas import tpu as pltpu
from jax.experimental.pallas import tpu_sc as plsc

N_NBR = 8
V_THRESH = 0.001
B, H, W = 4, 64, 96
N = H * W
NC, NS, L = 2, 16, 16
NW = NC * NS
QPW = B * N // NW
NVEC = QPW // L
CHUNK = 16
DONE = N_NBR * QPW

_dyg, _dxg = np.mgrid[-(H - 1):H, -(W - 1):W]
_dxf = _dxg.ravel().astype(np.int64)
_dyf = _dyg.ravel().astype(np.int64)
_order = np.lexsort((_dyf * W + _dxf, _dxf * _dxf + _dyf * _dyf))
_dxs, _dys = _dxf[_order], _dyf[_order]
NCAND = len(_dxs)
_pad = (-NCAND) % CHUNK
_words = ((_dys + 64) << 9) | (_dxs + 96)
_words = np.concatenate([_words, np.full(_pad, (255 << 9) | 511, np.int64)])
NCTAB = len(_words)
NCHUNKS = NCTAB // CHUNK
_TAB_NP = _words.astype(np.int32)

SEG0 = 2
_SEG_ENDS = [6, 30, NCHUNKS]

_mesh = plsc.VectorSubcoreMesh(core_axis_name="c", subcore_axis_name="s")


@functools.partial(
    pl.kernel,
    out_type=(
        jax.ShapeDtypeStruct((B, 2, N_NBR, N), jnp.float32),
        jax.ShapeDtypeStruct((B, N_NBR, N), jnp.int32),
    ),
    mesh=_mesh,
    compiler_params=pltpu.CompilerParams(needs_layout_passes=False),
    scratch_types=[
        pltpu.VMEM((NCTAB,), jnp.int32),
        pltpu.VMEM((N,), jnp.float32),
        pltpu.VMEM((N_NBR * QPW,), jnp.int32),
        pltpu.VMEM((N_NBR * QPW,), jnp.float32),
        pltpu.VMEM((N_NBR * QPW,), jnp.float32),
    ],
)
def _dist_sc(s_hbm, tab_hbm, of_hbm, args_hbm, tab_v, s_v, args_l, ofx_l, ofy_l):
    wid = lax.axis_index("s") * NC + lax.axis_index("c")
    b = wid // (NW // B)
    part = wid % (NW // B)
    qbase = part * QPW

    pltpu.sync_copy(tab_hbm, tab_v)
    pltpu.sync_copy(s_hbm.at[b], s_v)

    zero16 = jnp.zeros((L,), jnp.int32)

    def init_body(i, _):
        args_l[pl.ds(i * L, L)] = zero16
        return 0

    lax.fori_loop(0, N_NBR * QPW // L, init_body, 0)

    def vec_body(v, _):
        qloc = v * L + lax.iota(jnp.int32, L)
        q = qbase + qloc
        qy = (q * 2731) >> 18
        qx = q - qy * W

        def chunk_body(c, sidx):
            base_vec = jnp.full((L,), c * CHUNK, jnp.int32)
            for k in range(CHUNK):
                word = plsc.load_gather(tab_v, [base_vec + k])
                dxk = (word & 511) - 96
                dyk = (word >> 9) - 64
                nx = qx + dxk
                ny = qy + dyk
                inb = ((nx.astype(jnp.uint32) < W)
                       & (ny.astype(jnp.uint32) < H))
                j = ny * W + nx
                jc = jnp.where(inb, j, 0)
                sval = plsc.load_gather(s_v, [jc])
                found = inb & (sval > V_THRESH) & (sidx < DONE)
                plsc.store_scatter(args_l, [sidx], j, mask=found)
                sidx = sidx + jnp.where(found, QPW, 0)
            return sidx

        sidx = lax.fori_loop(0, SEG0, chunk_body, qloc)
        start = SEG0
        for end in _SEG_ENDS:
            left = jnp.max(jnp.where(sidx < DONE, 1, 0))
            stop = jnp.where(left > 0, end, start)
            sidx = lax.fori_loop(start, stop, chunk_body, sidx)
            start = end

        for k in range(N_NBR):
            jv = args_l[pl.ds(k * QPW + v * L, L)]
            jy = (jv * 2731) >> 18
            jx = jv - jy * W
            ofx_l[pl.ds(k * QPW + v * L, L)] = (jx - qx).astype(jnp.float32)
            ofy_l[pl.ds(k * QPW + v * L, L)] = (jy - qy).astype(jnp.float32)
        return 0

    lax.fori_loop(0, NVEC, vec_body, 0)

    for k in range(N_NBR):
        pltpu.sync_copy(args_l.at[pl.ds(k * QPW, QPW)],
                        args_hbm.at[b, k, pl.ds(qbase, QPW)])
        pltpu.sync_copy(ofx_l.at[pl.ds(k * QPW, QPW)],
                        of_hbm.at[b, 0, k, pl.ds(qbase, QPW)])
        pltpu.sync_copy(ofy_l.at[pl.ds(k * QPW, QPW)],
                        of_hbm.at[b, 1, k, pl.ds(qbase, QPW)])


@jax.jit
def kernel(S):
    Ofnum, args = _dist_sc(S.reshape(B, N), jnp.asarray(_TAB_NP))
    return (Ofnum, args)


# speedup vs baseline: 287.1837x; 1.7398x over previous
import functools
import numpy as np
import jax
import jax.numpy as jnp
from jax import lax
from jax.experimental import pallas as pl
from jax.experimental.pallas import tpu as pltpu
from jax.experimental.pallas import tpu_sc as plsc

N_NBR = 8
V_THRESH = 0.001
B, H, W = 4, 64, 96
N = H * W
NC, NS, L = 2, 16, 16
NW = NC * NS
QPW = B * N // NW
NVEC = QPW // L
CHUNK = 16

_dyg, _dxg = np.mgrid[-(H - 1):H, -(W - 1):W]
_dxf = _dxg.ravel().astype(np.int64)
_dyf = _dyg.ravel().astype(np.int64)
_order = np.lexsort((_dyf * W + _dxf, _dxf * _dxf + _dyf * _dyf))
_dxs, _dys = _dxf[_order], _dyf[_order]
NCAND = len(_dxs)
_pad = (-NCAND) % CHUNK
_words = ((_dys + 64) << 9) | (_dxs + 96)
_words = np.concatenate([_words, np.full(_pad, (255 << 9) | 511, np.int64)])
NCTAB = len(_words)
NCHUNKS = NCTAB // CHUNK
_TAB_NP = _words.astype(np.int32)

SEG0 = 2
_SEG_ENDS = [6, 30, NCHUNKS]

_mesh = plsc.VectorSubcoreMesh(core_axis_name="c", subcore_axis_name="s")


@functools.partial(
    pl.kernel,
    out_type=(
        jax.ShapeDtypeStruct((B, 2, N_NBR, N), jnp.float32),
        jax.ShapeDtypeStruct((B, N_NBR, N), jnp.int32),
    ),
    mesh=_mesh,
    compiler_params=pltpu.CompilerParams(needs_layout_passes=False),
    scratch_types=[
        pltpu.VMEM((NCTAB,), jnp.int32),
        pltpu.VMEM((N,), jnp.float32),
        pltpu.VMEM((N_NBR, QPW), jnp.int32),
        pltpu.VMEM((N_NBR, QPW), jnp.float32),
        pltpu.VMEM((N_NBR, QPW), jnp.float32),
    ],
)
def _dist_sc(s_hbm, tab_hbm, of_hbm, args_hbm, tab_v, s_v, args_l, ofx_l, ofy_l):
    wid = lax.axis_index("s") * NC + lax.axis_index("c")
    b = wid // (NW // B)
    part = wid % (NW // B)
    qbase = part * QPW

    pltpu.sync_copy(tab_hbm, tab_v)
    pltpu.sync_copy(s_hbm.at[b], s_v)

    def vec_body(v, _):
        qloc = v * L + lax.iota(jnp.int32, L)
        q = qbase + qloc
        qy = (q * 2731) >> 18
        qx = q - qy * W

        def chunk_body(c, cnt):
            base_vec = jnp.full((L,), c * CHUNK, jnp.int32)
            words = [plsc.load_gather(tab_v, [base_vec + k])
                     for k in range(CHUNK)]
            founds = []
            js = []
            for k in range(CHUNK):
                word = words[k]
                nx = qx + ((word & 511) - 96)
                ny = qy + ((word >> 9) - 64)
                inb = ((nx.astype(jnp.uint32) < W)
                       & (ny.astype(jnp.uint32) < H))
                j = ny * W + nx
                sval = plsc.load_gather(s_v, [jnp.where(inb, j, 0)])
                founds.append(inb & (sval > V_THRESH))
                js.append(j)
            for k in range(CHUNK):
                m = founds[k] & (cnt < N_NBR)
                plsc.store_scatter(args_l, [cnt, qloc], js[k], mask=m)
                cnt = cnt + jnp.where(founds[k], 1, 0)
            return cnt

        cnt = lax.fori_loop(0, SEG0, chunk_body, jnp.zeros((L,), jnp.int32))
        start = SEG0
        for end in _SEG_ENDS:
            left = jnp.max(jnp.where(cnt < N_NBR, 1, 0))
            stop = jnp.where(left > 0, end, start)
            cnt = lax.fori_loop(start, stop, chunk_body, cnt)
            start = end

        for k in range(N_NBR):
            jv = args_l[k, pl.ds(v * L, L)]
            jy = (jv * 2731) >> 18
            jx = jv - jy * W
            ofx_l[k, pl.ds(v * L, L)] = (jx - qx).astype(jnp.float32)
            ofy_l[k, pl.ds(v * L, L)] = (jy - qy).astype(jnp.float32)
        return 0

    lax.fori_loop(0, NVEC, vec_body, 0)

    pltpu.sync_copy(args_l, args_hbm.at[b, :, pl.ds(qbase, QPW)])
    pltpu.sync_copy(ofx_l, of_hbm.at[b, 0, :, pl.ds(qbase, QPW)])
    pltpu.sync_copy(ofy_l, of_hbm.at[b, 1, :, pl.ds(qbase, QPW)])


@jax.jit
def kernel(S):
    Ofnum, args = _dist_sc(S.reshape(B, N), jnp.asarray(_TAB_NP))
    return (Ofnum, args)
